# baseline (speedup 1.0000x reference)
import functools

import jax
import jax.numpy as jnp
from jax.experimental import pallas as pl
from jax.experimental.pallas import tpu as pltpu

N = 4096
F = 128
RB = 256
NB = N // RB

_bf = jnp.bfloat16


def _body(x_ref, l_ref, g1_ref, bt1_ref, w1_ref, b1_ref, g2_ref, bt2_ref,
          w2_ref, b2_ref, out_ref,
          lb, xbn, x1, y, y1):
    i = pl.program_id(0)

    def bn_affine(v, g_ref, bt_ref):
        mean = jnp.mean(v, axis=0, keepdims=True)
        var = jnp.mean(jnp.square(v), axis=0, keepdims=True) - jnp.square(mean)
        scale = g_ref[...] * jax.lax.rsqrt(var + 1e-5)
        shift = bt_ref[...] - mean * scale
        return v * scale + shift

    @pl.when(i < NB)
    def _load_pass():
        @pl.when(i == 0)
        def _():
            xbn[...] = bn_affine(x_ref[...], g1_ref, bt1_ref).astype(_bf)

        rows = pl.ds(i * RB, RB)
        lblk = l_ref[...].astype(_bf)
        lb[rows, :] = lblk
        x1[rows, :] = jnp.dot(lblk, xbn[...], preferred_element_type=jnp.float32).astype(_bf)

    @pl.when(i == NB)
    def _pass1():
        lx1v = jnp.dot(lb[...], x1[...], preferred_element_type=jnp.float32).astype(_bf)
        h = (jnp.dot(xbn[...], (w1_ref[0] - w1_ref[2]).astype(_bf), preferred_element_type=jnp.float32)
             + jnp.dot(x1[...], w1_ref[1].astype(_bf), preferred_element_type=jnp.float32)
             + jnp.dot(lx1v, (2.0 * w1_ref[2]).astype(_bf), preferred_element_type=jnp.float32)
             + b1_ref[...])
        out1 = jnp.maximum(h, 0.0)
        y[...] = bn_affine(out1, g2_ref, bt2_ref).astype(_bf)

    @pl.when(i == NB + 1)
    def _pass2():
        y1[...] = jnp.dot(lb[...], y[...], preferred_element_type=jnp.float32).astype(_bf)

    @pl.when(i == NB + 2)
    def _pass3():
        t = jnp.dot(lb[...], y1[...], preferred_element_type=jnp.float32).astype(_bf)
        out2 = (jnp.dot(y[...], (w2_ref[0] - w2_ref[2]).astype(_bf), preferred_element_type=jnp.float32)
                + jnp.dot(y1[...], w2_ref[1].astype(_bf), preferred_element_type=jnp.float32)
                + jnp.dot(t, (2.0 * w2_ref[2]).astype(_bf), preferred_element_type=jnp.float32)
                + b2_ref[...])
        res = xbn[...].astype(jnp.float32) + out2
        out_ref[...] = jnp.maximum(res, 0.0)


@functools.partial(jax.jit, static_argnames=("interpret",))
def _run(x, laplacian, g1, bt1, W1, b1, g2, bt2, W2, b2, interpret=False):
    full = pl.BlockSpec((N, F), lambda i: (0, 0))
    vec = pl.BlockSpec((1, F), lambda i: (0, 0))
    wspec = pl.BlockSpec(W1.shape, lambda i: (0, 0, 0))
    lspec = pl.BlockSpec((RB, N), lambda i: (jnp.minimum(i, NB - 1), 0))

    return pl.pallas_call(
        _body,
        grid=(NB + 3,),
        in_specs=[full, lspec, vec, vec, wspec, vec, vec, vec, wspec, vec],
        out_specs=full,
        out_shape=jax.ShapeDtypeStruct((N, F), jnp.float32),
        scratch_shapes=[
            pltpu.VMEM((N, N), _bf),
            pltpu.VMEM((N, F), _bf),
            pltpu.VMEM((N, F), _bf),
            pltpu.VMEM((N, F), _bf),
            pltpu.VMEM((N, F), _bf),
        ],
        compiler_params=pltpu.CompilerParams(
            dimension_semantics=("arbitrary",),
            vmem_limit_bytes=62 * 1024 * 1024,
        ),
        interpret=interpret,
    )(x, laplacian, g1, bt1, W1, b1, g2, bt2, W2, b2)


def kernel(x, laplacian, bn1_gamma, bn1_beta, W1, b1, bn2_gamma, bn2_beta,
           W2, b2):
    r = lambda v: v.reshape(1, F)
    return _run(x, laplacian, r(bn1_gamma), r(bn1_beta), W1, r(b1),
                r(bn2_gamma), r(bn2_beta), W2, r(b2))
